# Initial kernel scaffold; baseline (speedup 1.0000x reference)
import jax
import jax.numpy as jnp
from jax.experimental import pallas as pl


def kernel(x, pos, batch, seed_idx, W1, b1, W2, b2):
    raise NotImplementedError("write your pallas kernel here")


# speedup vs baseline: 11.1871x; 11.1871x over previous
import functools

import jax
import jax.numpy as jnp
from jax import lax
from jax.experimental import pallas as pl
from jax.experimental.pallas import tpu as pltpu
from jax.experimental.pallas import tpu_sc as plsc

N_PTS = 10000
N_PAD = 10240
N_SAMP = 2500
Q_PAD = 2560
K_NBR = 64
R2 = 0.2 * 0.2
D_FEAT = 128
HID = 128
OUT_D = 128
NEG_INF = float("-inf")
POS_INF = float("inf")


def _fps_body(posT_ref, idx_ref):
    pt = posT_ref[...]
    px, py, pz = pt[0], pt[1], pt[2]
    flat = (lax.broadcasted_iota(jnp.int32, (80, 128), 0) * 128
            + lax.broadcasted_iota(jnp.int32, (80, 128), 1))
    in_rng = flat < N_PTS
    flatq = (lax.broadcasted_iota(jnp.int32, (20, 128), 0) * 128
             + lax.broadcasted_iota(jnp.int32, (20, 128), 1))

    def dist_to(nxt):
        oh = flat == nxt
        qx = jnp.sum(jnp.where(oh, px, 0.0))
        qy = jnp.sum(jnp.where(oh, py, 0.0))
        qz = jnp.sum(jnp.where(oh, pz, 0.0))
        dx = px - qx
        dy = py - qy
        dz = pz - qz
        return dx * dx + dy * dy + dz * dz

    d0 = jnp.where(in_rng, dist_to(jnp.int32(0)), NEG_INF)
    acc0 = jnp.zeros((20, 128), jnp.int32)

    def body(i, state):
        dists, acc = state
        m = jnp.max(dists)
        nxt = jnp.min(jnp.where(dists == m, flat, jnp.int32(2 ** 30)))
        acc = jnp.where(flatq == i, nxt, acc)
        d = dist_to(nxt)
        return jnp.minimum(dists, d), acc

    _, acc = lax.fori_loop(1, N_SAMP, body, (d0, acc0))
    idx_ref[...] = acc


def _fps(posT_pad):
    return pl.pallas_call(
        _fps_body,
        out_shape=jax.ShapeDtypeStruct((20, 128), jnp.int32),
    )(posT_pad)


def _lin1_body(xc_ref, w_ref, b_ref, g_ref):
    g_ref[...] = (
        jnp.dot(xc_ref[...], w_ref[...], preferred_element_type=jnp.float32)
        + b_ref[...]
    )


def _lin1(xc_pad, w1c, b1):
    return pl.pallas_call(
        _lin1_body,
        grid=(5,),
        in_specs=[
            pl.BlockSpec((2000, 256), lambda i: (i, 0)),
            pl.BlockSpec((256, HID), lambda i: (0, 0)),
            pl.BlockSpec((1, HID), lambda i: (0, 0)),
        ],
        out_specs=pl.BlockSpec((2000, HID), lambda i: (i, 0)),
        out_shape=jax.ShapeDtypeStruct((N_PTS, HID), jnp.float32),
    )(xc_pad, w1c, b1.reshape(1, HID))


def _knn_body(q8_ref, p8t_ref, neigh_ref):
    q8 = q8_ref[...]
    p8t = p8t_ref[...]
    p2 = jnp.sum(p8t * p8t, axis=0, keepdims=True)
    q2 = jnp.sum(q8 * q8, axis=1, keepdims=True)
    qp = jnp.dot(q8, p8t, preferred_element_type=jnp.float32)
    d2 = jnp.maximum(q2 + p2 - 2.0 * qp, 0.0)
    colI = lax.broadcasted_iota(jnp.int32, (1, N_PAD), 1)
    ok = (d2 <= R2) & (colI < N_PTS)
    d = jnp.where(ok, d2, POS_INF)
    colK = lax.broadcasted_iota(jnp.int32, (1, K_NBR), 1)

    def body(k, state):
        t, li, acc = state
        elig = (d > t) | ((d == t) & (colI > li))
        dd = jnp.where(elig, d, POS_INF)
        m = jnp.min(dd, axis=1, keepdims=True)
        idx = jnp.min(jnp.where(dd == m, colI, jnp.int32(2 ** 30)),
                      axis=1, keepdims=True)
        found = m < POS_INF
        sel = jnp.where(found, idx, jnp.int32(-1))
        acc = jnp.where(colK == k, sel, acc)
        return m, jnp.where(found, idx, jnp.int32(2 ** 30)), acc

    t0 = jnp.full((128, 1), NEG_INF)
    li0 = jnp.full((128, 1), jnp.int32(-1))
    acc0 = jnp.full((128, K_NBR), jnp.int32(-1))
    _, _, acc = lax.fori_loop(0, K_NBR, body, (t0, li0, acc0))
    neigh_ref[...] = acc


def _knn(q8_pad, p8t):
    return pl.pallas_call(
        _knn_body,
        grid=(Q_PAD // 128,),
        in_specs=[
            pl.BlockSpec((128, 8), lambda i: (i, 0)),
            pl.BlockSpec((8, N_PAD), lambda i: (0, 0)),
        ],
        out_specs=pl.BlockSpec((128, K_NBR), lambda i: (i, 0)),
        out_shape=jax.ShapeDtypeStruct((Q_PAD, K_NBR), jnp.int32),
    )(q8_pad, p8t)


_B_EDGE = Q_PAD * K_NBR
_NW = 32
_B_PER_W = _B_EDGE // _NW
_CHUNK = 128
_N_CHUNK = _B_PER_W // _CHUNK


def _sc_gather_body(g_hbm, idx_hbm, out_hbm, idx_v, rows_v, sem):
    wid = lax.axis_index("s") * 2 + lax.axis_index("c")
    base = wid * _B_PER_W

    def chunk(c, carry):
        off = base + c * _CHUNK
        pltpu.sync_copy(idx_hbm.at[pl.ds(off, _CHUNK)], idx_v)
        pltpu.async_copy(g_hbm.at[idx_v], rows_v, sem).wait()
        pltpu.sync_copy(rows_v, out_hbm.at[pl.ds(off, _CHUNK)])
        return carry

    lax.fori_loop(0, _N_CHUNK, chunk, 0)


@functools.cache
def _sc_gather_kernel():
    return pl.kernel(
        _sc_gather_body,
        out_type=jax.ShapeDtypeStruct((_B_EDGE, HID), jnp.float32),
        mesh=plsc.VectorSubcoreMesh(core_axis_name="c", subcore_axis_name="s"),
        scratch_types=[
            pltpu.VMEM((_CHUNK,), jnp.int32),
            pltpu.VMEM((_CHUNK, HID), jnp.float32),
            pltpu.SemaphoreType.DMA,
        ],
    )


def _sc_gather(g, idxg):
    return _sc_gather_kernel()(g, idxg)


_QB = 8


def _mlp_body(cnt_ref, xg_ref, q8_ref, w1b_ref, w2_ref, b2_ref, out_ref):
    i = pl.program_id(0)
    nrows = _QB * K_NBR
    v = jnp.dot(q8_ref[...], w1b_ref[...],
                preferred_element_type=jnp.float32)
    rdiv = lax.broadcasted_iota(jnp.int32, (nrows, _QB), 0) // K_NBR
    expm = (rdiv == lax.broadcasted_iota(jnp.int32, (nrows, _QB), 1)
            ).astype(jnp.float32)
    vexp = jnp.dot(expm, v, preferred_element_type=jnp.float32)
    h1 = jnp.maximum(xg_ref[...] - vexp, 0.0)
    h2 = (jnp.dot(h1, w2_ref[...], preferred_element_type=jnp.float32)
          + b2_ref[...])
    rowI = lax.broadcasted_iota(jnp.int32, (nrows, OUT_D), 0)
    rowmod = rowI % K_NBR
    rowdiv = rowI // K_NBR
    cb = jnp.zeros((nrows, OUT_D), jnp.int32)
    for q in range(_QB):
        cb = jnp.where(rowdiv == q, cnt_ref[i * _QB + q], cb)
    hm = jnp.where(rowmod < cb, h2, NEG_INF)
    outI = lax.broadcasted_iota(jnp.int32, (_QB, OUT_D), 0)
    acc = jnp.full((_QB, OUT_D), NEG_INF)
    for q in range(_QB):
        mx = jnp.max(hm[q * K_NBR:(q + 1) * K_NBR, :], axis=0, keepdims=True)
        acc = jnp.where(outI == q, mx, acc)
    out_ref[...] = jnp.where(jnp.isfinite(acc), acc, 0.0)


def _mlp(counts, xg, q8_pad, w1b8, w2, b2):
    return pl.pallas_call(
        _mlp_body,
        grid=(Q_PAD // _QB,),
        in_specs=[
            pl.BlockSpec(memory_space=pltpu.SMEM),
            pl.BlockSpec((_QB * K_NBR, HID), lambda i: (i, 0)),
            pl.BlockSpec((_QB, 8), lambda i: (i, 0)),
            pl.BlockSpec((8, HID), lambda i: (0, 0)),
            pl.BlockSpec((HID, OUT_D), lambda i: (0, 0)),
            pl.BlockSpec((1, OUT_D), lambda i: (0, 0)),
        ],
        out_specs=pl.BlockSpec((_QB, OUT_D), lambda i: (i, 0)),
        out_shape=jax.ShapeDtypeStruct((Q_PAD, OUT_D), jnp.float32),
    )(counts, xg, q8_pad, w1b8, w2, b2.reshape(1, OUT_D))


def kernel(x, pos, batch, seed_idx, W1, b1, W2, b2):
    posT_pad = jnp.zeros((3, N_PAD), jnp.float32).at[:, :N_PTS].set(pos.T)
    idx = _fps(posT_pad.reshape(3, 80, 128)).reshape(Q_PAD)[:N_SAMP]

    pos_q = pos[idx]

    xc_pad = jnp.zeros((N_PTS, 256), jnp.float32)
    xc_pad = xc_pad.at[:, :D_FEAT].set(x).at[:, D_FEAT:D_FEAT + 3].set(pos)
    w1c = jnp.zeros((256, HID), jnp.float32).at[:D_FEAT + 3].set(W1)
    G = _lin1(xc_pad, w1c, b1)

    q8_pad = jnp.zeros((Q_PAD, 8), jnp.float32).at[:N_SAMP, :3].set(pos_q)
    p8t = posT_pad.reshape(3, N_PAD)
    p8t = jnp.zeros((8, N_PAD), jnp.float32).at[:3].set(p8t)
    neigh = _knn(q8_pad, p8t)

    idxg = jnp.maximum(neigh, 0).reshape(_B_EDGE)
    Xg = _sc_gather(G, idxg)

    counts = jnp.sum((neigh >= 0).astype(jnp.int32), axis=1)
    w1b8 = jnp.zeros((8, HID), jnp.float32).at[:3].set(W1[D_FEAT:])
    out = _mlp(counts, Xg, q8_pad, w1b8, W2, b2)[:N_SAMP]

    return out, pos_q, batch[idx], seed_idx[idx]
